# Initial kernel scaffold; baseline (speedup 1.0000x reference)
import jax
import jax.numpy as jnp
from jax.experimental import pallas as pl


def kernel(x, weight):
    raise NotImplementedError("write your pallas kernel here")


# speedup vs baseline: 142.9258x; 142.9258x over previous
import functools

import jax
import jax.numpy as jnp
from jax import lax
from jax.experimental import pallas as pl
from jax.experimental.pallas import tpu as pltpu
from jax.experimental.pallas import tpu_sc as plsc

NC = 2
NS = 16
L = 16
NW = NC * NS
TOTAL = 16384 * 100
PER_W = TOTAL // NW
VECS = PER_W // L


def _sc_body(x_hbm, w_hbm, out_hbm, x_v, w_v, s_v, acc_v, stage_v, shared):
    cid = lax.axis_index("c")
    sid = lax.axis_index("s")
    wid = cid * NS + sid
    base = wid * PER_W

    pltpu.sync_copy(x_hbm.at[pl.ds(base, PER_W)], x_v)
    pltpu.sync_copy(w_hbm, w_v)

    s = w_v[0, :]
    for c in range(1, 2 * L):
        s = s + w_v[c, :]
    s_v[...] = s

    def body(i, acc):
        idx = x_v[pl.ds(i * L, L)]
        return acc + plsc.load_gather(s_v, [idx])

    acc = lax.fori_loop(0, VECS, body, jnp.zeros((L,), jnp.float32))
    acc_v[...] = acc

    pltpu.sync_copy(acc_v, out_hbm.at[wid])


_sc_call = functools.partial(
    pl.kernel,
    out_type=jax.ShapeDtypeStruct((NW, L), jnp.float32),
    mesh=plsc.VectorSubcoreMesh(core_axis_name="c", subcore_axis_name="s"),
    compiler_params=pltpu.CompilerParams(needs_layout_passes=False),
    scratch_types=[
        pltpu.VMEM((PER_W,), jnp.int32),
        pltpu.VMEM((2 * L, L), jnp.float32),
        pltpu.VMEM((L,), jnp.float32),
        pltpu.VMEM((L,), jnp.float32),
        pltpu.VMEM((NS, L), jnp.float32),
        pltpu.VMEM_SHARED((NS, L), jnp.float32),
    ],
)


def kernel(x, weight):
    x_flat = x.reshape(-1).astype(jnp.int32)
    w_t = jnp.zeros((2 * L, L), jnp.float32).at[:20, :10].set(
        weight.astype(jnp.float32).T)
    out = _sc_call(_sc_body)(x_flat, w_t)
    return out.sum()
